# baseline (speedup 1.0000x reference)
import functools

import jax
import jax.numpy as jnp
from jax import lax
from jax.experimental import pallas as pl
from jax.experimental.pallas import tpu as pltpu
from jax.experimental.pallas import tpu_sc as plsc

HD = 64
INTD = 21
L = 16
NC, NS = 2, 16
NW = NC * NS
C = 128


def _proj_small_body(e_int, e_test, e_tag, w_int, w_test, w_tag, b,
                     p_int, p_test, p_tag):
    p_int[...] = jnp.dot(e_int[...], w_int[...],
                         preferred_element_type=jnp.float32) + b[...]
    p_test[...] = jnp.dot(e_test[...], w_test[...],
                          preferred_element_type=jnp.float32)
    p_tag[...] = jnp.dot(e_tag[...], w_tag[...],
                         preferred_element_type=jnp.float32)


def _proj_q_body(e_q, w_q, p_q):
    p_q[...] = jnp.dot(e_q[...], w_q[...], preferred_element_type=jnp.float32)


def _gather_sum_body(n_per_w, n_chunks,
                     i_int, i_test, i_q, i_tag,
                     p_int, p_test, p_q, p_tag,
                     out_hbm,
                     idx0, idx1, idx2, idx3,
                     b0, b1, b2, b3, ob,
                     sem0, sem1, sem2, sem3):
    wid = lax.axis_index("s") * NC + lax.axis_index("c")
    wbase = wid * n_per_w

    def chunk_body(ci, carry):
        base = wbase + ci * C
        pltpu.sync_copy(i_int.at[pl.ds(base, C)], idx0)
        pltpu.sync_copy(i_test.at[pl.ds(base, C)], idx1)
        pltpu.sync_copy(i_q.at[pl.ds(base, C)], idx2)
        pltpu.sync_copy(i_tag.at[pl.ds(base, C)], idx3)
        cp0 = pltpu.async_copy(p_int.at[idx0], b0, sem0)
        cp1 = pltpu.async_copy(p_test.at[idx1], b1, sem1)
        cp2 = pltpu.async_copy(p_q.at[idx2], b2, sem2)
        cp3 = pltpu.async_copy(p_tag.at[idx3], b3, sem3)
        cp0.wait()
        cp1.wait()
        cp2.wait()
        cp3.wait()

        def add_body(r, c2):
            for j in range(HD // L):
                s = pl.ds(j * L, L)
                ob[r, s] = b0[r, s] + b1[r, s] + b2[r, s] + b3[r, s]
            return c2

        lax.fori_loop(0, C, add_body, 0, unroll=2)
        pltpu.sync_copy(ob, out_hbm.at[pl.ds(base, C)])
        return carry

    lax.fori_loop(0, n_chunks, chunk_body, 0)


def kernel(test, question, tag, correct, mask, interaction,
           emb_interaction, emb_test, emb_question, emb_tag,
           W_comb, b_comb):
    B, S = interaction.shape
    N = B * S
    n_per_w = N // NW
    n_chunks = n_per_w // C

    w_int = W_comb[0 * INTD:1 * INTD]
    w_test = W_comb[1 * INTD:2 * INTD]
    w_q = W_comb[2 * INTD:3 * INTD]
    w_tag = W_comb[3 * INTD:4 * INTD]
    b2d = b_comb.reshape(1, HD)

    n_int = emb_interaction.shape[0]
    n_test = emb_test.shape[0]
    n_q = emb_question.shape[0]
    n_tag = emb_tag.shape[0]

    p_int, p_test, p_tag = pl.pallas_call(
        _proj_small_body,
        out_shape=[
            jax.ShapeDtypeStruct((n_int, HD), jnp.float32),
            jax.ShapeDtypeStruct((n_test, HD), jnp.float32),
            jax.ShapeDtypeStruct((n_tag, HD), jnp.float32),
        ],
    )(emb_interaction, emb_test, emb_tag, w_int, w_test, w_tag, b2d)

    RQ = 8192
    grid_q = (n_q + RQ - 1) // RQ
    p_q = pl.pallas_call(
        _proj_q_body,
        grid=(grid_q,),
        in_specs=[
            pl.BlockSpec((RQ, INTD), lambda i: (i, 0)),
            pl.BlockSpec((INTD, HD), lambda i: (0, 0)),
        ],
        out_specs=pl.BlockSpec((RQ, HD), lambda i: (i, 0)),
        out_shape=jax.ShapeDtypeStruct((n_q, HD), jnp.float32),
    )(emb_question, w_q)

    mesh = plsc.VectorSubcoreMesh(core_axis_name="c", subcore_axis_name="s")
    sc = functools.partial(
        pl.kernel,
        out_type=jax.ShapeDtypeStruct((N, HD), jnp.float32),
        mesh=mesh,
        compiler_params=pltpu.CompilerParams(use_tc_tiling_on_sc=False),
        scratch_types=[
            pltpu.VMEM((C,), jnp.int32),
            pltpu.VMEM((C,), jnp.int32),
            pltpu.VMEM((C,), jnp.int32),
            pltpu.VMEM((C,), jnp.int32),
            pltpu.VMEM((C, HD), jnp.float32),
            pltpu.VMEM((C, HD), jnp.float32),
            pltpu.VMEM((C, HD), jnp.float32),
            pltpu.VMEM((C, HD), jnp.float32),
            pltpu.VMEM((C, HD), jnp.float32),
            pltpu.SemaphoreType.DMA,
            pltpu.SemaphoreType.DMA,
            pltpu.SemaphoreType.DMA,
            pltpu.SemaphoreType.DMA,
        ],
    )(functools.partial(_gather_sum_body, n_per_w, n_chunks))

    out_flat = sc(interaction.reshape(N), test.reshape(N),
                  question.reshape(N), tag.reshape(N),
                  p_int, p_test, p_q, p_tag)

    X = out_flat.reshape(B, S, HD)
    return (X, B)


# speedup vs baseline: 1.0035x; 1.0035x over previous
import functools

import jax
import jax.numpy as jnp
from jax import lax
from jax.experimental import pallas as pl
from jax.experimental.pallas import tpu as pltpu
from jax.experimental.pallas import tpu_sc as plsc

HD = 64
INTD = 21
L = 16
NC, NS = 2, 16
NW = NC * NS
C = 128


def _proj_small_body(e_int, e_test, e_tag, w_int, w_test, w_tag, b,
                     p_int, p_test, p_tag):
    p_int[...] = jnp.dot(e_int[...], w_int[...],
                         preferred_element_type=jnp.float32) + b[...]
    p_test[...] = jnp.dot(e_test[...], w_test[...],
                          preferred_element_type=jnp.float32)
    p_tag[...] = jnp.dot(e_tag[...], w_tag[...],
                         preferred_element_type=jnp.float32)


def _proj_q_body(e_q, w_q, p_q):
    p_q[...] = jnp.dot(e_q[...], w_q[...], preferred_element_type=jnp.float32)


IBC = 25
IB = IBC * C


def _gather_sum_body(n_per_w, n_blocks,
                     i_int, i_test, i_q, i_tag,
                     p_int, p_test, p_q, p_tag,
                     out_hbm,
                     idx0, idx1, idx2, idx3,
                     gb, ob,
                     sem_g0, sem_g1, sem_ob0, sem_ob1):
    wid = lax.axis_index("s") * NC + lax.axis_index("c")
    wbase = wid * n_per_w
    sem_g = (sem_g0, sem_g1)
    sem_ob = (sem_ob0, sem_ob1)

    def fire(j, slot):
        off = j * C
        return [
            pltpu.async_copy(p_int.at[idx0.at[pl.ds(off, C)]], gb[0][slot], sem_g[slot]),
            pltpu.async_copy(p_test.at[idx1.at[pl.ds(off, C)]], gb[1][slot], sem_g[slot]),
            pltpu.async_copy(p_q.at[idx2.at[pl.ds(off, C)]], gb[2][slot], sem_g[slot]),
            pltpu.async_copy(p_tag.at[idx3.at[pl.ds(off, C)]], gb[3][slot], sem_g[slot]),
        ]

    def block_body(blk, carry):
        boff = wbase + blk * IB
        pltpu.sync_copy(i_int.at[pl.ds(boff, IB)], idx0)
        pltpu.sync_copy(i_test.at[pl.ds(boff, IB)], idx1)
        pltpu.sync_copy(i_q.at[pl.ds(boff, IB)], idx2)
        pltpu.sync_copy(i_tag.at[pl.ds(boff, IB)], idx3)

        descs = fire(0, 0)
        ob_descs = [None, None]
        for j in range(IBC):
            slot = j % 2
            descs_next = fire(j + 1, 1 - slot) if j + 1 < IBC else None
            for d in descs:
                d.wait()
            if ob_descs[slot] is not None:
                ob_descs[slot].wait()
            b0, b1, b2, b3 = (gb[t][slot] for t in range(4))
            obuf = ob[slot]

            @plsc.parallel_loop(0, C, step=1, unroll=4)
            def add_body(r):
                for cg in range(HD // L):
                    s = pl.ds(cg * L, L)
                    obuf[r, s] = b0[r, s] + b1[r, s] + b2[r, s] + b3[r, s]

            base = boff + j * C
            ob_descs[slot] = pltpu.async_copy(
                obuf, out_hbm.at[pl.ds(base, C)], sem_ob[slot])
            descs = descs_next
        for d in ob_descs:
            if d is not None:
                d.wait()
        return carry

    lax.fori_loop(0, n_blocks, block_body, 0)


def kernel(test, question, tag, correct, mask, interaction,
           emb_interaction, emb_test, emb_question, emb_tag,
           W_comb, b_comb):
    B, S = interaction.shape
    N = B * S
    n_per_w = N // NW
    n_blocks = n_per_w // IB

    w_int = W_comb[0 * INTD:1 * INTD]
    w_test = W_comb[1 * INTD:2 * INTD]
    w_q = W_comb[2 * INTD:3 * INTD]
    w_tag = W_comb[3 * INTD:4 * INTD]
    b2d = b_comb.reshape(1, HD)

    n_int = emb_interaction.shape[0]
    n_test = emb_test.shape[0]
    n_q = emb_question.shape[0]
    n_tag = emb_tag.shape[0]

    p_int, p_test, p_tag = pl.pallas_call(
        _proj_small_body,
        out_shape=[
            jax.ShapeDtypeStruct((n_int, HD), jnp.float32),
            jax.ShapeDtypeStruct((n_test, HD), jnp.float32),
            jax.ShapeDtypeStruct((n_tag, HD), jnp.float32),
        ],
    )(emb_interaction, emb_test, emb_tag, w_int, w_test, w_tag, b2d)

    RQ = 8192
    grid_q = (n_q + RQ - 1) // RQ
    p_q = pl.pallas_call(
        _proj_q_body,
        grid=(grid_q,),
        in_specs=[
            pl.BlockSpec((RQ, INTD), lambda i: (i, 0)),
            pl.BlockSpec((INTD, HD), lambda i: (0, 0)),
        ],
        out_specs=pl.BlockSpec((RQ, HD), lambda i: (i, 0)),
        out_shape=jax.ShapeDtypeStruct((n_q, HD), jnp.float32),
    )(emb_question, w_q)

    mesh = plsc.VectorSubcoreMesh(core_axis_name="c", subcore_axis_name="s")
    sc = functools.partial(
        pl.kernel,
        out_type=jax.ShapeDtypeStruct((N, HD), jnp.float32),
        mesh=mesh,
        compiler_params=pltpu.CompilerParams(use_tc_tiling_on_sc=False),
        scratch_types=[
            pltpu.VMEM((IB,), jnp.int32),
            pltpu.VMEM((IB,), jnp.int32),
            pltpu.VMEM((IB,), jnp.int32),
            pltpu.VMEM((IB,), jnp.int32),
            [[pltpu.VMEM((C, HD), jnp.float32) for _ in range(2)]
             for _ in range(4)],
            [pltpu.VMEM((C, HD), jnp.float32) for _ in range(2)],
            pltpu.SemaphoreType.DMA,
            pltpu.SemaphoreType.DMA,
            pltpu.SemaphoreType.DMA,
            pltpu.SemaphoreType.DMA,
        ],
    )(functools.partial(_gather_sum_body, n_per_w, n_blocks))

    out_flat = sc(interaction.reshape(N), test.reshape(N),
                  question.reshape(N), tag.reshape(N),
                  p_int, p_test, p_q, p_tag)

    X = out_flat.reshape(B, S, HD)
    return (X, B)
